# Initial kernel scaffold; baseline (speedup 1.0000x reference)
import functools

import jax
import jax.numpy as jnp
from jax import lax
from jax.experimental import pallas as pl
from jax.experimental.pallas import tpu as pltpu
from jax.experimental.pallas import tpu_sc as plsc

N = 10000
E = 320000
D = 128
H2 = 64

GROUP = 128
NG = E // GROUP
NC = 2
NS = 16
NW = NC * NS
ROWS_PER_SUB = N // NS
DEGW = 16

_sc_mesh = functools.partial(
    plsc.VectorSubcoreMesh, core_axis_name="c", subcore_axis_name="s")


def _sc_scatter_body(z, srcs, dsts, zeros2d, msg_out, deg_out,
                     acc, dacc, src_v, dst_v, rows_v, ones_v, gsem,
                     with_deg):
    cid = lax.axis_index("c")
    sid = lax.axis_index("s")
    wid = sid * NC + cid
    r0 = sid * ROWS_PER_SUB

    pltpu.sync_copy(zeros2d.at[pl.ds(r0, ROWS_PER_SUB)],
                    acc.at[pl.ds(r0, ROWS_PER_SUB)])
    if with_deg:
        pltpu.sync_copy(zeros2d.at[pl.ds(r0, ROWS_PER_SUB), pl.ds(0, DEGW)],
                        dacc.at[pl.ds(r0, ROWS_PER_SUB)])

        def _fill(i, carry):
            ones_v[i, :] = jnp.ones((DEGW,), jnp.float32)
            return carry
        lax.fori_loop(0, GROUP, _fill, 0)
    plsc.subcore_barrier()

    niter = (NG - wid + NW - 1) // NW

    def _body(i, carry):
        base = (wid + i * NW) * GROUP
        pltpu.sync_copy(srcs.at[pl.ds(base, GROUP)], src_v)
        pltpu.sync_copy(dsts.at[pl.ds(base, GROUP)], dst_v.at[0])
        pltpu.async_copy(z.at[src_v], rows_v, gsem).wait()
        pltpu.sync_copy(rows_v, acc.at[dst_v.at[0]], add=True)
        if with_deg:
            pltpu.sync_copy(ones_v, dacc.at[dst_v.at[0]], add=True)
        return carry
    lax.fori_loop(0, niter, _body, 0)
    plsc.subcore_barrier()

    pltpu.sync_copy(acc.at[pl.ds(r0, ROWS_PER_SUB)],
                    msg_out.at[cid, pl.ds(r0, ROWS_PER_SUB)])
    if with_deg:
        pltpu.sync_copy(dacc.at[pl.ds(r0, ROWS_PER_SUB)],
                        deg_out.at[cid, pl.ds(r0, ROWS_PER_SUB)])


@functools.partial(
    pl.kernel,
    mesh=_sc_mesh(),
    out_type=[jax.ShapeDtypeStruct((NC, N, D), jnp.float32),
              jax.ShapeDtypeStruct((NC, N, DEGW), jnp.float32)],
    scratch_types=[
        pltpu.VMEM_SHARED((N, D), jnp.float32),
        pltpu.VMEM_SHARED((N, DEGW), jnp.float32),
        pltpu.VMEM((GROUP,), jnp.int32),
        pltpu.VMEM((1, GROUP), jnp.int32),
        pltpu.VMEM((GROUP, D), jnp.float32),
        pltpu.VMEM((GROUP, DEGW), jnp.float32),
        pltpu.SemaphoreType.DMA,
    ],
)
def _sc_scatter_deg(z, srcs, dsts, zeros2d, msg_out, deg_out,
                    acc, dacc, src_v, dst_v, rows_v, ones_v, gsem):
    _sc_scatter_body(z, srcs, dsts, zeros2d, msg_out, deg_out,
                     acc, dacc, src_v, dst_v, rows_v, ones_v, gsem,
                     with_deg=True)


@functools.partial(
    pl.kernel,
    mesh=_sc_mesh(),
    out_type=[jax.ShapeDtypeStruct((NC, N, D), jnp.float32)],
    scratch_types=[
        pltpu.VMEM_SHARED((N, D), jnp.float32),
        pltpu.VMEM((GROUP,), jnp.int32),
        pltpu.VMEM((1, GROUP), jnp.int32),
        pltpu.VMEM((GROUP, D), jnp.float32),
        pltpu.SemaphoreType.DMA,
    ],
)
def _sc_scatter(z, srcs, dsts, zeros2d, msg_out,
                acc, src_v, dst_v, rows_v, gsem):
    _sc_scatter_body(z, srcs, dsts, zeros2d, msg_out, None,
                     acc, None, src_v, dst_v, rows_v, None, gsem,
                     with_deg=False)


def _tc_matmul_body(x_ref, w_ref, o_ref):
    o_ref[...] = lax.dot_general(
        x_ref[...], w_ref[...], (((1,), (1,)), ((), ())),
        preferred_element_type=jnp.float32)


_tc_matmul = pl.pallas_call(
    _tc_matmul_body,
    out_shape=jax.ShapeDtypeStruct((N, D), jnp.float32),
)


def _tc_mid_body(msg_ref, deg_ref, z_ref, b_ref, w_ref, o_ref):
    msg = msg_ref[0] + msg_ref[1]
    deg = deg_ref[0, :, 0:1] + deg_ref[1, :, 0:1]
    z = z_ref[...]
    agg = jnp.where(deg > 0.0, msg / jnp.maximum(deg, 1.0), z)
    h = jnp.maximum(agg + b_ref[...], 0.0)
    o_ref[...] = lax.dot_general(
        h, w_ref[...], (((1,), (1,)), ((), ())),
        preferred_element_type=jnp.float32)


_tc_mid = pl.pallas_call(
    _tc_mid_body,
    out_shape=jax.ShapeDtypeStruct((N, D), jnp.float32),
)


def _tc_post_body(msg_ref, deg_ref, z_ref, b_ref, wr1_ref, br1_ref,
                  wr2_ref, br2_ref, o_ref):
    msg = msg_ref[0] + msg_ref[1]
    deg = deg_ref[0, :, 0:1] + deg_ref[1, :, 0:1]
    z = z_ref[...]
    agg = jnp.where(deg > 0.0, msg / jnp.maximum(deg, 1.0), z)
    h = jnp.maximum(agg + b_ref[...], 0.0)
    hg = jnp.mean(h, axis=0, keepdims=True)
    hidden = lax.dot_general(
        hg, wr1_ref[...], (((1,), (1,)), ((), ())),
        preferred_element_type=jnp.float32) + br1_ref[...]
    out = lax.dot_general(
        hidden, wr2_ref[...], (((1,), (1,)), ((), ())),
        preferred_element_type=jnp.float32) + br2_ref[...]
    o_ref[...] = out


_tc_post = pl.pallas_call(
    _tc_post_body,
    out_shape=jax.ShapeDtypeStruct((1, 1), jnp.float32),
)


def kernel(x, edge_index, W1, b1, W2, b2, Wr1, br1, Wr2, br2):
    src = edge_index[0]
    dst = edge_index[1]
    zeros2d = jnp.zeros((N, D), jnp.float32)

    z1 = _tc_matmul(x, W1)
    msg1, deg = _sc_scatter_deg(z1, src, dst, zeros2d)
    z2 = _tc_mid(msg1, deg, z1, b1.reshape(1, D), W2)
    (msg2,) = _sc_scatter(z2, src, dst, zeros2d)
    out = _tc_post(msg2, deg, z2, b2.reshape(1, D), Wr1, br1.reshape(1, H2),
                   Wr2, br2.reshape(1, 1))
    return out


# speedup vs baseline: 5.3277x; 5.3277x over previous
import functools

import jax
import jax.numpy as jnp
from jax import lax
from jax.experimental import pallas as pl
from jax.experimental.pallas import tpu as pltpu
from jax.experimental.pallas import tpu_sc as plsc

N = 10000
E = 320000
D = 128
H2 = 64

GROUP = 128
NG = E // GROUP
NC = 2
NS = 16
NW = NC * NS
NPAD = 10240
ROWS_PER_SUB = NPAD // NS


def _sc_mesh():
    return plsc.VectorSubcoreMesh(
        core_axis_name="c", subcore_axis_name="s",
        num_cores=NC, num_subcores=NS)


def _worker_ids():
    cid = lax.axis_index("c")
    sid = lax.axis_index("s")
    return cid, sid, sid * NC + cid


@functools.cache
def _sc_scatter():
    @functools.partial(
        pl.kernel,
        mesh=_sc_mesh(),
        out_type=[jax.ShapeDtypeStruct((NC, NPAD, D), jnp.float32)],
        scratch_types=[
            pltpu.VMEM_SHARED((NPAD, D), jnp.float32),
            pltpu.VMEM((GROUP,), jnp.int32),
            pltpu.VMEM((1, GROUP), jnp.int32),
            pltpu.VMEM((GROUP, D), jnp.float32),
            pltpu.SemaphoreType.DMA,
        ],
    )
    def body(z, srcs, dsts, zeros_m, msg_out, acc, src_v, dst_v, rows_v, gsem):
        cid, sid, wid = _worker_ids()
        r0 = sid * ROWS_PER_SUB
        pltpu.sync_copy(zeros_m.at[pl.ds(r0, ROWS_PER_SUB)],
                        acc.at[pl.ds(r0, ROWS_PER_SUB)])
        plsc.subcore_barrier()

        niter = (NG - wid + NW - 1) // NW

        def _body(i, carry):
            base = (wid + i * NW) * GROUP
            pltpu.sync_copy(srcs.at[pl.ds(base, GROUP)], src_v)
            pltpu.sync_copy(dsts.at[pl.ds(base, GROUP)], dst_v.at[0])
            pltpu.async_copy(z.at[src_v], rows_v, gsem).wait()
            pltpu.sync_copy(rows_v, acc.at[dst_v.at[0]], add=True)
            return carry
        lax.fori_loop(0, niter, _body, 0)
        plsc.subcore_barrier()

        pltpu.sync_copy(acc.at[pl.ds(r0, ROWS_PER_SUB)],
                        msg_out.at[cid, pl.ds(r0, ROWS_PER_SUB)])
    return body


@functools.cache
def _sc_degree():
    @functools.partial(
        pl.kernel,
        mesh=_sc_mesh(),
        out_type=[jax.ShapeDtypeStruct((NC, NPAD, D), jnp.float32)],
        scratch_types=[
            pltpu.VMEM_SHARED((NPAD, D), jnp.float32),
            pltpu.VMEM((1, GROUP), jnp.int32),
            pltpu.VMEM((GROUP, D), jnp.float32),
        ],
    )
    def body(dsts, zeros_m, ones_m, deg_out, acc, dst_v, ones_v):
        cid, sid, wid = _worker_ids()
        r0 = sid * ROWS_PER_SUB
        pltpu.sync_copy(zeros_m.at[pl.ds(r0, ROWS_PER_SUB)],
                        acc.at[pl.ds(r0, ROWS_PER_SUB)])
        pltpu.sync_copy(ones_m, ones_v)
        plsc.subcore_barrier()

        niter = (NG - wid + NW - 1) // NW

        def _body(i, carry):
            base = (wid + i * NW) * GROUP
            pltpu.sync_copy(dsts.at[pl.ds(base, GROUP)], dst_v.at[0])
            pltpu.sync_copy(ones_v, acc.at[dst_v.at[0]], add=True)
            return carry
        lax.fori_loop(0, niter, _body, 0)
        plsc.subcore_barrier()

        pltpu.sync_copy(acc.at[pl.ds(r0, ROWS_PER_SUB)],
                        deg_out.at[cid, pl.ds(r0, ROWS_PER_SUB)])
    return body


def _tc_matmul_body(x_ref, w_ref, o_ref):
    o_ref[...] = lax.dot_general(
        x_ref[...], w_ref[...], (((1,), (1,)), ((), ())),
        preferred_element_type=jnp.float32)


_tc_matmul = pl.pallas_call(
    _tc_matmul_body,
    out_shape=jax.ShapeDtypeStruct((N, D), jnp.float32),
)


def _agg_relu(msg_ref, deg_ref, z_ref, b_ref):
    msg = msg_ref[0, :N] + msg_ref[1, :N]
    deg = deg_ref[0, :N] + deg_ref[1, :N]
    z = z_ref[...]
    agg = jnp.where(deg > 0.0, msg / jnp.maximum(deg, 1.0), z)
    return jnp.maximum(agg + b_ref[...], 0.0)


def _tc_mid_body(msg_ref, deg_ref, z_ref, b_ref, w_ref, o_ref):
    h = _agg_relu(msg_ref, deg_ref, z_ref, b_ref)
    o_ref[...] = lax.dot_general(
        h, w_ref[...], (((1,), (1,)), ((), ())),
        preferred_element_type=jnp.float32)


_tc_mid = pl.pallas_call(
    _tc_mid_body,
    out_shape=jax.ShapeDtypeStruct((N, D), jnp.float32),
)


def _tc_post_body(msg_ref, deg_ref, z_ref, b_ref, wr1_ref, br1_ref,
                  wr2_ref, br2_ref, o_ref):
    h = _agg_relu(msg_ref, deg_ref, z_ref, b_ref)
    hg = jnp.mean(h, axis=0, keepdims=True)
    hidden = lax.dot_general(
        hg, wr1_ref[...], (((1,), (1,)), ((), ())),
        preferred_element_type=jnp.float32) + br1_ref[...]
    o_ref[0, 0] = jnp.sum(hidden * wr2_ref[...]) + br2_ref[0]


_tc_post = pl.pallas_call(
    _tc_post_body,
    in_specs=[
        pl.BlockSpec(memory_space=pltpu.VMEM),
        pl.BlockSpec(memory_space=pltpu.VMEM),
        pl.BlockSpec(memory_space=pltpu.VMEM),
        pl.BlockSpec(memory_space=pltpu.VMEM),
        pl.BlockSpec(memory_space=pltpu.VMEM),
        pl.BlockSpec(memory_space=pltpu.VMEM),
        pl.BlockSpec(memory_space=pltpu.VMEM),
        pl.BlockSpec(memory_space=pltpu.SMEM),
    ],
    out_specs=pl.BlockSpec(memory_space=pltpu.SMEM),
    out_shape=jax.ShapeDtypeStruct((1, 1), jnp.float32),
)


def kernel(x, edge_index, W1, b1, W2, b2, Wr1, br1, Wr2, br2):
    src = edge_index[0]
    dst = edge_index[1]
    zeros_m = jnp.zeros((NPAD, D), jnp.float32)
    ones_m = jnp.ones((GROUP, D), jnp.float32)

    z1 = _tc_matmul(x, W1)
    (deg,) = _sc_degree()(dst, zeros_m, ones_m)
    (msg1,) = _sc_scatter()(z1, src, dst, zeros_m)
    z2 = _tc_mid(msg1, deg, z1, b1.reshape(1, D), W2)
    (msg2,) = _sc_scatter()(z2, src, dst, zeros_m)
    out = _tc_post(msg2, deg, z2, b2.reshape(1, D), Wr1, br1.reshape(1, H2),
                   Wr2, br2)
    return out


# speedup vs baseline: 9.2658x; 1.7392x over previous
import functools

import jax
import jax.numpy as jnp
from jax import lax
from jax.experimental import pallas as pl
from jax.experimental.pallas import tpu as pltpu
from jax.experimental.pallas import tpu_sc as plsc

N = 10000
E = 320000
D = 128
H2 = 64

GROUP = 128
NG = E // GROUP
NC = 2
NS = 16
NW = NC * NS
NPAD = 10240
ROWS_PER_SUB = NPAD // NS
GPT = 80
NGPAD = NW * GPT


def _sc_mesh():
    return plsc.VectorSubcoreMesh(
        core_axis_name="c", subcore_axis_name="s",
        num_cores=NC, num_subcores=NS)


def _worker_ids():
    cid = lax.axis_index("c")
    sid = lax.axis_index("s")
    return cid, sid, sid * NC + cid


@functools.cache
def _sc_scatter():
    @functools.partial(
        pl.kernel,
        mesh=_sc_mesh(),
        out_type=[jax.ShapeDtypeStruct((NC, NPAD, D), jnp.float32)],
        scratch_types=[
            pltpu.VMEM_SHARED((NPAD, D), jnp.float32),
            pltpu.VMEM((2, GROUP), jnp.int32),
            pltpu.VMEM((2, GROUP), jnp.int32),
            pltpu.VMEM((2, GROUP, D), jnp.float32),
            pltpu.SemaphoreType.DMA,
            pltpu.SemaphoreType.DMA,
            pltpu.SemaphoreType.DMA,
        ],
    )
    def body(z, srcs, dsts, zeros_m, msg_out,
             acc, src_v, dst_v, rows, gsem0, gsem1, isem):
        cid, sid, wid = _worker_ids()
        r0 = sid * ROWS_PER_SUB
        pltpu.sync_copy(zeros_m.at[pl.ds(r0, ROWS_PER_SUB)],
                        acc.at[pl.ds(r0, ROWS_PER_SUB)])
        plsc.subcore_barrier()

        niter = (NG - wid + NW - 1) // NW

        gsems = (gsem0, gsem1)

        def _fetch(i, slot):
            base = (wid + i * NW) * GROUP
            pltpu.async_copy(srcs.at[pl.ds(base, GROUP)], src_v.at[slot],
                             isem)
            pltpu.async_copy(dsts.at[pl.ds(base, GROUP)], dst_v.at[slot],
                             isem)
            pltpu.make_async_copy(srcs.at[pl.ds(0, GROUP)], src_v.at[slot],
                                  isem).wait()
            pltpu.make_async_copy(dsts.at[pl.ds(0, GROUP)], dst_v.at[slot],
                                  isem).wait()
            pltpu.async_copy(z.at[src_v.at[slot]], rows.at[slot],
                             gsems[slot])

        def _consume(slot):
            pltpu.make_async_copy(z.at[src_v.at[slot]], rows.at[slot],
                                  gsems[slot]).wait()
            pltpu.sync_copy(rows.at[slot], acc.at[dst_v.at[slot]], add=True)

        @pl.when(niter > 0)
        def _():
            _fetch(0, 0)

        def _pair(p, carry):
            i0 = 2 * p

            @pl.when(i0 + 1 < niter)
            def _():
                _fetch(i0 + 1, 1)
            _consume(0)

            @pl.when(i0 + 2 < niter)
            def _():
                _fetch(i0 + 2, 0)

            @pl.when(i0 + 1 < niter)
            def _():
                _consume(1)
            return carry

        lax.fori_loop(0, (niter + 1) // 2, _pair, 0)
        plsc.subcore_barrier()

        pltpu.sync_copy(acc.at[pl.ds(r0, ROWS_PER_SUB)],
                        msg_out.at[cid, pl.ds(r0, ROWS_PER_SUB)])
    return body


@functools.cache
def _sc_degree():
    @functools.partial(
        pl.kernel,
        mesh=_sc_mesh(),
        out_type=[jax.ShapeDtypeStruct((NC, NPAD, D), jnp.float32)],
        scratch_types=[
            pltpu.VMEM_SHARED((NPAD, D), jnp.float32),
            pltpu.VMEM((GPT, GROUP), jnp.int32),
            pltpu.VMEM((GROUP, D), jnp.float32),
            pltpu.SemaphoreType.DMA,
        ],
    )
    def body(dsts2d, zeros_m, ones_m, deg_out, acc, dst_buf, ones_v, ssem):
        cid, sid, wid = _worker_ids()
        r0 = sid * ROWS_PER_SUB
        g0 = wid * GPT
        pltpu.sync_copy(zeros_m.at[pl.ds(r0, ROWS_PER_SUB)],
                        acc.at[pl.ds(r0, ROWS_PER_SUB)])
        pltpu.sync_copy(ones_m, ones_v)
        pltpu.sync_copy(dsts2d.at[pl.ds(g0, GPT)], dst_buf)
        plsc.subcore_barrier()

        cnt = jnp.minimum(jnp.maximum(NG - g0, 0), GPT)
        KD = 16

        def _fire(i, carry):
            pltpu.async_copy(ones_v, acc.at[dst_buf.at[i]], ssem, add=True)
            return carry

        def _drain(i, carry):
            pltpu.make_async_copy(ones_v, acc.at[dst_buf.at[0]], ssem).wait()
            return carry

        def _block(b, carry):
            i0 = b * KD
            n = jnp.minimum(cnt - i0, KD)
            lax.fori_loop(i0, i0 + n, _fire, 0)
            lax.fori_loop(0, n, _drain, 0)
            return carry
        lax.fori_loop(0, (cnt + KD - 1) // KD, _block, 0)
        plsc.subcore_barrier()

        pltpu.sync_copy(acc.at[pl.ds(r0, ROWS_PER_SUB)],
                        deg_out.at[cid, pl.ds(r0, ROWS_PER_SUB)])
    return body


def _tc_matmul_body(x_ref, w_ref, o_ref):
    o_ref[...] = lax.dot_general(
        x_ref[...], w_ref[...], (((1,), (1,)), ((), ())),
        preferred_element_type=jnp.float32)


_tc_matmul = pl.pallas_call(
    _tc_matmul_body,
    out_shape=jax.ShapeDtypeStruct((N, D), jnp.float32),
)


def _agg_relu(msg_ref, deg_ref, z_ref, b_ref):
    msg = msg_ref[0, :N] + msg_ref[1, :N]
    deg = deg_ref[0, :N] + deg_ref[1, :N]
    z = z_ref[...]
    agg = jnp.where(deg > 0.0, msg / jnp.maximum(deg, 1.0), z)
    return jnp.maximum(agg + b_ref[...], 0.0)


def _tc_mid_body(msg_ref, deg_ref, z_ref, b_ref, w_ref, o_ref):
    h = _agg_relu(msg_ref, deg_ref, z_ref, b_ref)
    o_ref[...] = lax.dot_general(
        h, w_ref[...], (((1,), (1,)), ((), ())),
        preferred_element_type=jnp.float32)


_tc_mid = pl.pallas_call(
    _tc_mid_body,
    out_shape=jax.ShapeDtypeStruct((N, D), jnp.float32),
)


def _tc_post_body(msg_ref, deg_ref, z_ref, b_ref, wr1_ref, br1_ref,
                  wr2_ref, br2_ref, o_ref):
    h = _agg_relu(msg_ref, deg_ref, z_ref, b_ref)
    hg = jnp.mean(h, axis=0, keepdims=True)
    hidden = lax.dot_general(
        hg, wr1_ref[...], (((1,), (1,)), ((), ())),
        preferred_element_type=jnp.float32) + br1_ref[...]
    o_ref[0, 0] = jnp.sum(hidden * wr2_ref[...]) + br2_ref[0]


_tc_post = pl.pallas_call(
    _tc_post_body,
    in_specs=[
        pl.BlockSpec(memory_space=pltpu.VMEM),
        pl.BlockSpec(memory_space=pltpu.VMEM),
        pl.BlockSpec(memory_space=pltpu.VMEM),
        pl.BlockSpec(memory_space=pltpu.VMEM),
        pl.BlockSpec(memory_space=pltpu.VMEM),
        pl.BlockSpec(memory_space=pltpu.VMEM),
        pl.BlockSpec(memory_space=pltpu.VMEM),
        pl.BlockSpec(memory_space=pltpu.SMEM),
    ],
    out_specs=pl.BlockSpec(memory_space=pltpu.SMEM),
    out_shape=jax.ShapeDtypeStruct((1, 1), jnp.float32),
)


def kernel(x, edge_index, W1, b1, W2, b2, Wr1, br1, Wr2, br2):
    src = edge_index[0]
    dst = edge_index[1]
    pad = jnp.zeros((NGPAD - NG, GROUP), jnp.int32)
    dst2d = jnp.concatenate([dst.reshape(NG, GROUP), pad], axis=0)
    zeros_m = jnp.zeros((NPAD, D), jnp.float32)
    ones_m = jnp.ones((GROUP, D), jnp.float32)

    z1 = _tc_matmul(x, W1)
    (deg,) = _sc_degree()(dst2d, zeros_m, ones_m)
    (msg1,) = _sc_scatter()(z1, src, dst, zeros_m)
    z2 = _tc_mid(msg1, deg, z1, b1.reshape(1, D), W2)
    (msg2,) = _sc_scatter()(z2, src, dst, zeros_m)
    out = _tc_post(msg2, deg, z2, b2.reshape(1, D), Wr1, br1.reshape(1, H2),
                   Wr2, br2)
    return out


# speedup vs baseline: 9.4901x; 1.0242x over previous
import functools

import jax
import jax.numpy as jnp
from jax import lax
from jax.experimental import pallas as pl
from jax.experimental.pallas import tpu as pltpu
from jax.experimental.pallas import tpu_sc as plsc

N = 10000
E = 320000
D = 128
H2 = 64

GROUP = 128
NG = E // GROUP
NC = 2
NS = 16
NW = NC * NS
NPAD = 10240
ROWS_PER_SUB = NPAD // NS
GPT = 80
NGPAD = NW * GPT


def _sc_mesh():
    return plsc.VectorSubcoreMesh(
        core_axis_name="c", subcore_axis_name="s",
        num_cores=NC, num_subcores=NS)


def _worker_ids():
    cid = lax.axis_index("c")
    sid = lax.axis_index("s")
    return cid, sid, sid * NC + cid


def _zero_acc(zeros_m, acc, r0):
    pltpu.sync_copy(zeros_m.at[pl.ds(r0, ROWS_PER_SUB)],
                    acc.at[pl.ds(r0, ROWS_PER_SUB)])


def _msg_pipeline(z, srcs, dsts, acc, src_v, dst_v, rows,
                  gsems, ssems, isem, wid):
    niter = (NG - wid + NW - 1) // NW

    def _fetch(i, slot):
        base = (wid + i * NW) * GROUP
        pltpu.async_copy(srcs.at[pl.ds(base, GROUP)], src_v.at[slot], isem)
        pltpu.async_copy(dsts.at[pl.ds(base, GROUP)], dst_v.at[slot], isem)
        pltpu.make_async_copy(srcs.at[pl.ds(0, GROUP)], src_v.at[slot],
                              isem).wait()
        pltpu.make_async_copy(dsts.at[pl.ds(0, GROUP)], dst_v.at[slot],
                              isem).wait()
        pltpu.async_copy(z.at[src_v.at[slot]], rows.at[slot], gsems[slot])

    def _consume(slot):
        pltpu.make_async_copy(z.at[src_v.at[slot]], rows.at[slot],
                              gsems[slot]).wait()
        pltpu.sync_copy(rows.at[slot], acc.at[dst_v.at[slot]], add=True)

    @pl.when(niter > 0)
    def _():
        _fetch(0, 0)

    def _pair(p, carry):
        i0 = 2 * p

        @pl.when(i0 + 1 < niter)
        def _():
            _fetch(i0 + 1, 1)
        _consume(0)

        @pl.when(i0 + 2 < niter)
        def _():
            _fetch(i0 + 2, 0)

        @pl.when(i0 + 1 < niter)
        def _():
            _consume(1)
        return carry

    lax.fori_loop(0, (niter + 1) // 2, _pair, 0)


def _deg_phase(dsts2d, acc, dst_buf, ones_v, ssem, g0):
    cnt = jnp.minimum(jnp.maximum(NG - g0, 0), GPT)

    def _fire(i, carry):
        pltpu.sync_copy(ones_v, acc.at[dst_buf.at[i]], add=True)
        return carry

    lax.fori_loop(0, cnt, _fire, 0)


_SC_SCRATCH = [
    pltpu.VMEM_SHARED((NPAD, D), jnp.float32),
    pltpu.VMEM((2, GROUP), jnp.int32),
    pltpu.VMEM((2, GROUP), jnp.int32),
    pltpu.VMEM((2, GROUP, D), jnp.float32),
    pltpu.SemaphoreType.DMA,
    pltpu.SemaphoreType.DMA,
    pltpu.SemaphoreType.DMA,
    pltpu.SemaphoreType.DMA,
    pltpu.SemaphoreType.DMA,
]


@functools.cache
def _sc_layer1():
    @functools.partial(
        pl.kernel,
        mesh=_sc_mesh(),
        out_type=[jax.ShapeDtypeStruct((NC, NPAD, D), jnp.float32),
                  jax.ShapeDtypeStruct((NC, NPAD, D), jnp.float32)],
        scratch_types=_SC_SCRATCH + [
            pltpu.VMEM((GPT, GROUP), jnp.int32),
        ],
    )
    def body(z, srcs, dsts, dsts2d, zeros_m, ones_m, msg_out, deg_out,
             acc, src_v, dst_v, rows, gsem0, gsem1, ssem0, ssem1, isem,
             dst_buf):
        cid, sid, wid = _worker_ids()
        r0 = sid * ROWS_PER_SUB
        g0 = wid * GPT
        _zero_acc(zeros_m, acc, r0)
        pltpu.sync_copy(ones_m, rows.at[1])
        pltpu.sync_copy(dsts2d.at[pl.ds(g0, GPT)], dst_buf)
        plsc.subcore_barrier()

        _deg_phase(dsts2d, acc, dst_buf, rows.at[1], ssem0, g0)
        plsc.subcore_barrier()

        pltpu.sync_copy(acc.at[pl.ds(r0, ROWS_PER_SUB)],
                        deg_out.at[cid, pl.ds(r0, ROWS_PER_SUB)])
        _zero_acc(zeros_m, acc, r0)
        plsc.subcore_barrier()

        _msg_pipeline(z, srcs, dsts, acc, src_v, dst_v, rows,
                      (gsem0, gsem1), (ssem0, ssem1), isem, wid)
        plsc.subcore_barrier()

        pltpu.sync_copy(acc.at[pl.ds(r0, ROWS_PER_SUB)],
                        msg_out.at[cid, pl.ds(r0, ROWS_PER_SUB)])
    return body


@functools.cache
def _sc_scatter():
    @functools.partial(
        pl.kernel,
        mesh=_sc_mesh(),
        out_type=[jax.ShapeDtypeStruct((NC, NPAD, D), jnp.float32)],
        scratch_types=list(_SC_SCRATCH),
    )
    def body(z, srcs, dsts, zeros_m, msg_out,
             acc, src_v, dst_v, rows, gsem0, gsem1, ssem0, ssem1, isem):
        cid, sid, wid = _worker_ids()
        r0 = sid * ROWS_PER_SUB
        _zero_acc(zeros_m, acc, r0)
        plsc.subcore_barrier()

        _msg_pipeline(z, srcs, dsts, acc, src_v, dst_v, rows,
                      (gsem0, gsem1), (ssem0, ssem1), isem, wid)
        plsc.subcore_barrier()

        pltpu.sync_copy(acc.at[pl.ds(r0, ROWS_PER_SUB)],
                        msg_out.at[cid, pl.ds(r0, ROWS_PER_SUB)])
    return body


def _layer(msg_ref, deg_ref, h_ref, w_ref, b_ref):
    msg = msg_ref[0, :N] + msg_ref[1, :N]
    deg = deg_ref[0, :N] + deg_ref[1, :N]
    h = h_ref[...]
    agg = jnp.where(deg > 0.0, msg / jnp.maximum(deg, 1.0), h)
    out = lax.dot_general(
        agg, w_ref[...], (((1,), (1,)), ((), ())),
        preferred_element_type=jnp.float32) + b_ref[...]
    return jnp.maximum(out, 0.0)


def _tc_mid_body(msg_ref, deg_ref, x_ref, w_ref, b_ref, o_ref):
    o_ref[...] = _layer(msg_ref, deg_ref, x_ref, w_ref, b_ref)


_tc_mid = pl.pallas_call(
    _tc_mid_body,
    out_shape=jax.ShapeDtypeStruct((N, D), jnp.float32),
)


def _tc_post_body(msg_ref, deg_ref, h1_ref, w_ref, b_ref, wr1_ref, br1_ref,
                  wr2_ref, br2_ref, o_ref):
    h = _layer(msg_ref, deg_ref, h1_ref, w_ref, b_ref)
    hg = jnp.mean(h, axis=0, keepdims=True)
    hidden = lax.dot_general(
        hg, wr1_ref[...], (((1,), (1,)), ((), ())),
        preferred_element_type=jnp.float32) + br1_ref[...]
    o_ref[0, 0] = jnp.sum(hidden * wr2_ref[...]) + br2_ref[0]


_tc_post = pl.pallas_call(
    _tc_post_body,
    in_specs=[
        pl.BlockSpec(memory_space=pltpu.VMEM),
        pl.BlockSpec(memory_space=pltpu.VMEM),
        pl.BlockSpec(memory_space=pltpu.VMEM),
        pl.BlockSpec(memory_space=pltpu.VMEM),
        pl.BlockSpec(memory_space=pltpu.VMEM),
        pl.BlockSpec(memory_space=pltpu.VMEM),
        pl.BlockSpec(memory_space=pltpu.VMEM),
        pl.BlockSpec(memory_space=pltpu.VMEM),
        pl.BlockSpec(memory_space=pltpu.SMEM),
    ],
    out_specs=pl.BlockSpec(memory_space=pltpu.SMEM),
    out_shape=jax.ShapeDtypeStruct((1, 1), jnp.float32),
)


def kernel(x, edge_index, W1, b1, W2, b2, Wr1, br1, Wr2, br2):
    src = edge_index[0]
    dst = edge_index[1]
    pad = jnp.zeros((NGPAD - NG, GROUP), jnp.int32)
    dst2d = jnp.concatenate([dst.reshape(NG, GROUP), pad], axis=0)
    zeros_m = jnp.zeros((NPAD, D), jnp.float32)
    ones_m = jnp.ones((GROUP, D), jnp.float32)

    msg1, deg = _sc_layer1()(x, src, dst, dst2d, zeros_m, ones_m)
    h1 = _tc_mid(msg1, deg, x, W1, b1.reshape(1, D))
    (msg2,) = _sc_scatter()(h1, src, dst, zeros_m)
    out = _tc_post(msg2, deg, h1, W2, b2.reshape(1, D), Wr1,
                   br1.reshape(1, H2), Wr2, br2)
    return out
